# baseline (speedup 1.0000x reference)
import jax
import jax.numpy as jnp
from jax.experimental import pallas as pl
from jax.experimental.pallas import tpu as pltpu


def _body(alpha_ref, p_ref, m_ref, sb_ref, o_ref):
    a = alpha_ref[0]
    o_ref[...] = jnp.where(m_ref[...], sb_ref[...] - a * p_ref[...],
                           jnp.float32(-1e9))


def kernel(pairwise_tti, combined_mask, state_bias, alpha, bias):
    b, t, i, j = pairwise_tti.shape
    n = b * t
    m = i * j
    p2 = pairwise_tti.reshape(n, m)
    m2 = combined_mask.reshape(n, m)
    sb = (state_bias + bias).reshape(n, 1)
    R = 64
    grid = (n // R,)
    out = pl.pallas_call(
        _body,
        grid_spec=pltpu.PrefetchScalarGridSpec(
            num_scalar_prefetch=1,
            grid=grid,
            in_specs=[
                pl.BlockSpec((R, m), lambda g, *_: (g, 0)),
                pl.BlockSpec((R, m), lambda g, *_: (g, 0)),
                pl.BlockSpec((R, 1), lambda g, *_: (g, 0)),
            ],
            out_specs=pl.BlockSpec((R, m), lambda g, *_: (g, 0)),
        ),
        out_shape=jax.ShapeDtypeStruct((n, m), jnp.float32),
    )(alpha.reshape(1), p2, m2, sb)
    return out.reshape(b, t, m)


# speedup vs baseline: 1.3897x; 1.3897x over previous
import jax
import jax.numpy as jnp
from jax.experimental import pallas as pl
from jax.experimental.pallas import tpu as pltpu

_TG = 8


def _body(alpha_ref, p_ref, m_ref, sb_ref, o_ref):
    a = alpha_ref[0]
    v = sb_ref[...].reshape(1, _TG, 1, 1) - a * p_ref[...]
    v = jnp.where(m_ref[...], v, jnp.float32(-1e9))
    o_ref[...] = v.reshape(1, _TG, p_ref.shape[2] * p_ref.shape[3])


def kernel(pairwise_tti, combined_mask, state_bias, alpha, bias):
    b, t, i, j = pairwise_tti.shape
    m = i * j
    sb = (state_bias + bias).reshape(b * t, 1)
    grid = (b, t // _TG)
    out = pl.pallas_call(
        _body,
        grid_spec=pltpu.PrefetchScalarGridSpec(
            num_scalar_prefetch=1,
            grid=grid,
            in_specs=[
                pl.BlockSpec((1, _TG, i, j), lambda bb, g, *_: (bb, g, 0, 0)),
                pl.BlockSpec((1, _TG, i, j), lambda bb, g, *_: (bb, g, 0, 0)),
                pl.BlockSpec((_TG, 1), lambda bb, g, *_: (bb * (64 // _TG) + g, 0)),
            ],
            out_specs=pl.BlockSpec((1, _TG, m), lambda bb, g, *_: (bb, g, 0)),
        ),
        out_shape=jax.ShapeDtypeStruct((b, t, m), jnp.float32),
    )(alpha.reshape(1), pairwise_tti, combined_mask, sb)
    return out


# speedup vs baseline: 2.0688x; 1.4886x over previous
import jax
import jax.numpy as jnp
from jax.experimental import pallas as pl
from jax.experimental.pallas import tpu as pltpu

_TG = 32


def _body(alpha_ref, p_ref, m_ref, sb_ref, o_ref):
    a = alpha_ref[0]
    v = sb_ref[...].reshape(1, _TG, 1, 1) - a * p_ref[...]
    v = jnp.where(m_ref[...], v, jnp.float32(-1e9))
    o_ref[...] = v.reshape(1, _TG, p_ref.shape[2] * p_ref.shape[3])


def kernel(pairwise_tti, combined_mask, state_bias, alpha, bias):
    b, t, i, j = pairwise_tti.shape
    m = i * j
    sb = (state_bias + bias).reshape(b * t, 1)
    grid = (b, t // _TG)
    out = pl.pallas_call(
        _body,
        grid_spec=pltpu.PrefetchScalarGridSpec(
            num_scalar_prefetch=1,
            grid=grid,
            in_specs=[
                pl.BlockSpec((1, _TG, i, j), lambda bb, g, *_: (bb, g, 0, 0)),
                pl.BlockSpec((1, _TG, i, j), lambda bb, g, *_: (bb, g, 0, 0)),
                pl.BlockSpec((_TG, 1), lambda bb, g, *_: (bb * (64 // _TG) + g, 0)),
            ],
            out_specs=pl.BlockSpec((1, _TG, m), lambda bb, g, *_: (bb, g, 0)),
        ),
        out_shape=jax.ShapeDtypeStruct((b, t, m), jnp.float32),
    )(alpha.reshape(1), pairwise_tti, combined_mask, sb)
    return out


# speedup vs baseline: 2.1485x; 1.0385x over previous
import jax
import jax.numpy as jnp
from jax.experimental import pallas as pl
from jax.experimental.pallas import tpu as pltpu

_TG = 64


def _body(alpha_ref, p_ref, m_ref, sb_ref, o_ref):
    a = alpha_ref[0]
    v = sb_ref[...].reshape(1, _TG, 1, 1) - a * p_ref[...]
    v = jnp.where(m_ref[...], v, jnp.float32(-1e9))
    o_ref[...] = v.reshape(1, _TG, p_ref.shape[2] * p_ref.shape[3])


def kernel(pairwise_tti, combined_mask, state_bias, alpha, bias):
    b, t, i, j = pairwise_tti.shape
    m = i * j
    sb = (state_bias + bias).reshape(b * t, 1)
    grid = (b, t // _TG)
    out = pl.pallas_call(
        _body,
        grid_spec=pltpu.PrefetchScalarGridSpec(
            num_scalar_prefetch=1,
            grid=grid,
            in_specs=[
                pl.BlockSpec((1, _TG, i, j), lambda bb, g, *_: (bb, g, 0, 0)),
                pl.BlockSpec((1, _TG, i, j), lambda bb, g, *_: (bb, g, 0, 0)),
                pl.BlockSpec((_TG, 1), lambda bb, g, *_: (bb * (64 // _TG) + g, 0)),
            ],
            out_specs=pl.BlockSpec((1, _TG, m), lambda bb, g, *_: (bb, g, 0)),
        ),
        out_shape=jax.ShapeDtypeStruct((b, t, m), jnp.float32),
    )(alpha.reshape(1), pairwise_tti, combined_mask, sb)
    return out
